# baseline (speedup 1.0000x reference)
import functools

import jax
import jax.numpy as jnp
import numpy as np
from jax.experimental import pallas as pl
from jax.experimental.pallas import tpu as pltpu

N = 2000
R = 400
NB = N // R
CH = 128
TOPK = 200
SIG2 = np.float32(0.1 ** 2)
NEG = np.float32(-np.inf)


def _fcg_sv_body(src_r, tgt_r, srcT, tgtT, fcg_out, sv_out, vals):
    i = pl.program_id(0)

    def dist(own, allT):
        d2 = None
        for c in range(3):
            a = own[:, c:c + 1]
            b = allT[c:c + 1, :]
            e = a - b
            d2 = e * e if d2 is None else d2 + e * e
        return jnp.sqrt(jnp.maximum(d2, 0.0))

    pd = dist(src_r[...], srcT[...]) - dist(tgt_r[...], tgtT[...])
    fcg = jnp.maximum(1.0 - (pd * pd) / SIG2, 0.0)
    rows = i * R + jax.lax.broadcasted_iota(jnp.int32, (R, N), 0)
    cols = jax.lax.broadcasted_iota(jnp.int32, (R, N), 1)
    fcg = fcg * (1.0 - (rows == cols).astype(jnp.float32))
    fcg_out[...] = fcg
    vals[...] = fcg
    sv_out[...] = jnp.zeros((R, TOPK), jnp.float32)

    lane = jax.lax.broadcasted_iota(jnp.int32, (R, TOPK), 1)
    colid = jax.lax.broadcasted_iota(jnp.int32, (R, N), 1)

    def body(t, _):
        v = vals[...]
        m = jnp.max(v, axis=1, keepdims=True)
        fi = jnp.min(jnp.where(v == m, colid, N), axis=1, keepdims=True)
        vals[...] = jnp.where(colid == fi, NEG, v)
        mv = jnp.maximum(m, 0.0)
        sv_out[...] += jnp.where(lane == t, mv, 0.0)
        return 0

    jax.lax.fori_loop(0, TOPK, body, 0)


def _fcg_sv(srcp, tgtp, srcT, tgtT):
    return pl.pallas_call(
        _fcg_sv_body,
        grid=(NB,),
        in_specs=[
            pl.BlockSpec((R, 3), lambda i: (i, 0)),
            pl.BlockSpec((R, 3), lambda i: (i, 0)),
            pl.BlockSpec((3, N), lambda i: (0, 0)),
            pl.BlockSpec((3, N), lambda i: (0, 0)),
        ],
        out_specs=[
            pl.BlockSpec((R, N), lambda i: (i, 0)),
            pl.BlockSpec((R, TOPK), lambda i: (i, 0)),
        ],
        out_shape=[
            jax.ShapeDtypeStruct((N, N), jnp.float32),
            jax.ShapeDtypeStruct((N, TOPK), jnp.float32),
        ],
        scratch_shapes=[pltpu.VMEM((R, N), jnp.float32)],
        compiler_params=pltpu.CompilerParams(dimension_semantics=("parallel",)),
    )(srcp, tgtp, srcT, tgtT)


def _h_body(thr, fa, fb, h_out, nnz_out):
    t = thr[0, 0]
    a = fa[...]
    a = jnp.where(a < t, 0.0, a)
    b = fb[...]
    b = jnp.where(b < t, 0.0, b)
    h = jnp.dot(a, b, preferred_element_type=jnp.float32) * a
    h_out[...] = h
    cnt = jnp.sum((h > 0).astype(jnp.float32))

    @pl.when(pl.program_id(0) == 0)
    def _():
        nnz_out[...] = jnp.zeros_like(nnz_out)

    nnz_out[...] += cnt


def _h_matmul(fcg, thresh):
    return pl.pallas_call(
        _h_body,
        grid=(NB,),
        in_specs=[
            pl.BlockSpec(memory_space=pltpu.SMEM),
            pl.BlockSpec((R, N), lambda i: (i, 0)),
            pl.BlockSpec((N, N), lambda i: (0, 0)),
        ],
        out_specs=[
            pl.BlockSpec((R, N), lambda i: (i, 0)),
            pl.BlockSpec((8, 128), lambda i: (0, 0)),
        ],
        out_shape=[
            jax.ShapeDtypeStruct((N, N), jnp.float32),
            jax.ShapeDtypeStruct((8, 128), jnp.float32),
        ],
    )(thresh, fcg, fcg)


def _adj_row(hblk, flag, i):
    a = (hblk > 0).astype(jnp.float32)
    rows = i * R + jax.lax.broadcasted_iota(jnp.int32, (R, N), 0)
    cols = jax.lax.broadcasted_iota(jnp.int32, (R, N), 1)
    eye = (rows == cols).astype(jnp.float32)
    return jnp.where(flag > 0, a, eye)


def _esum_body(flag, hb, xfull, w_in, b_in, esum_out, deg_out):
    i = pl.program_id(0)
    a = _adj_row(hb[...], flag[0, 0], i)
    h = jnp.maximum(jnp.dot(xfull[...], w_in[...],
                            preferred_element_type=jnp.float32)
                    + b_in[0:1, :], 0.0)
    esum_out[...] = jnp.dot(a, h, preferred_element_type=jnp.float32)
    deg_out[...] = jnp.sum(a, axis=1, keepdims=True) + jnp.zeros((R, CH), jnp.float32)


def _esum(flag, H, xp, w_inp, b_in8):
    return pl.pallas_call(
        _esum_body,
        grid=(NB,),
        in_specs=[
            pl.BlockSpec(memory_space=pltpu.SMEM),
            pl.BlockSpec((R, N), lambda i: (i, 0)),
            pl.BlockSpec((N, 8), lambda i: (0, 0)),
            pl.BlockSpec((8, CH), lambda i: (0, 0)),
            pl.BlockSpec((8, CH), lambda i: (0, 0)),
        ],
        out_specs=[
            pl.BlockSpec((R, CH), lambda i: (i, 0)),
            pl.BlockSpec((R, CH), lambda i: (i, 0)),
        ],
        out_shape=[
            jax.ShapeDtypeStruct((N, CH), jnp.float32),
            jax.ShapeDtypeStruct((N, CH), jnp.float32),
        ],
        compiler_params=pltpu.CompilerParams(dimension_semantics=("parallel",)),
    )(flag, H, xp, w_inp, b_in8)


def _msum_body(flag, hb, esum, deg, msum_out):
    i = pl.program_id(0)
    a = _adj_row(hb[...], flag[0, 0], i)
    emean = esum[...] / jnp.maximum(deg[...][:, 0:1], 1.0)
    msum_out[...] = jnp.dot(a, emean, preferred_element_type=jnp.float32)


def _msum(flag, H, esum, deg):
    return pl.pallas_call(
        _msum_body,
        grid=(NB,),
        in_specs=[
            pl.BlockSpec(memory_space=pltpu.SMEM),
            pl.BlockSpec((R, N), lambda i: (i, 0)),
            pl.BlockSpec((N, CH), lambda i: (0, 0)),
            pl.BlockSpec((N, CH), lambda i: (0, 0)),
        ],
        out_specs=pl.BlockSpec((R, CH), lambda i: (i, 0)),
        out_shape=jax.ShapeDtypeStruct((N, CH), jnp.float32),
        compiler_params=pltpu.CompilerParams(dimension_semantics=("parallel",)),
    )(flag, H, esum, deg)


def _mlp_body(bo, xi, w_in, b_in, msumi, degi, w_hid, b_hid, w_outp,
              h2_out, log_out):
    hk = jnp.maximum(jnp.dot(xi[...], w_in[...],
                             preferred_element_type=jnp.float32)
                     + b_in[0:1, :], 0.0)
    m = msumi[...] / jnp.maximum(degi[...][:, 0:1], 1.0)
    h2 = jnp.maximum(jnp.dot(hk + m, w_hid[...],
                             preferred_element_type=jnp.float32)
                     + b_hid[0:1, :], 0.0)
    h2_out[...] = h2
    log_out[...] = jnp.dot(h2, w_outp[...],
                           preferred_element_type=jnp.float32) + bo[0, 0]


def _mlp(b_out, xp, w_inp, b_in8, msum, deg, w_hid, b_hid8, w_outp):
    return pl.pallas_call(
        _mlp_body,
        grid=(NB,),
        in_specs=[
            pl.BlockSpec(memory_space=pltpu.SMEM),
            pl.BlockSpec((R, 8), lambda i: (i, 0)),
            pl.BlockSpec((8, CH), lambda i: (0, 0)),
            pl.BlockSpec((8, CH), lambda i: (0, 0)),
            pl.BlockSpec((R, CH), lambda i: (i, 0)),
            pl.BlockSpec((R, CH), lambda i: (i, 0)),
            pl.BlockSpec((CH, CH), lambda i: (0, 0)),
            pl.BlockSpec((8, CH), lambda i: (0, 0)),
            pl.BlockSpec((CH, CH), lambda i: (0, 0)),
        ],
        out_specs=[
            pl.BlockSpec((R, CH), lambda i: (i, 0)),
            pl.BlockSpec((R, CH), lambda i: (i, 0)),
        ],
        out_shape=[
            jax.ShapeDtypeStruct((N, CH), jnp.float32),
            jax.ShapeDtypeStruct((N, CH), jnp.float32),
        ],
        compiler_params=pltpu.CompilerParams(dimension_semantics=("parallel",)),
    )(b_out, xp, w_inp, b_in8, msum, deg, w_hid, b_hid8, w_outp)


def _m_body(s2, ai, bfull, m_out):
    i = pl.program_id(0)
    acc = jax.lax.dot_general(ai[...], bfull[...], (((1,), (1,)), ((), ())),
                              preferred_element_type=jnp.float32)
    v = jnp.clip(1.0 - (1.0 - acc) / s2[0, 0], 0.0, 1.0)
    rows = i * R + jax.lax.broadcasted_iota(jnp.int32, (R, N), 0)
    cols = jax.lax.broadcasted_iota(jnp.int32, (R, N), 1)
    m_out[...] = v * (1.0 - (rows == cols).astype(jnp.float32))


def _m_matmul(sig2, h2):
    return pl.pallas_call(
        _m_body,
        grid=(NB,),
        in_specs=[
            pl.BlockSpec(memory_space=pltpu.SMEM),
            pl.BlockSpec((R, CH), lambda i: (i, 0)),
            pl.BlockSpec((N, CH), lambda i: (0, 0)),
        ],
        out_specs=pl.BlockSpec((R, N), lambda i: (i, 0)),
        out_shape=jax.ShapeDtypeStruct((N, N), jnp.float32),
        compiler_params=pltpu.CompilerParams(dimension_semantics=("parallel",)),
    )(sig2, h2, h2)


def _deg_nbr_body(hb, conf8, d_out, nbr_out):
    hblk = hb[...]
    mg = (hblk + hblk) > 1.0
    d = jnp.sum(mg.astype(jnp.float32), axis=1, keepdims=True)
    d_out[...] = d + jnp.zeros((R, CH), jnp.float32)
    c = conf8[...][0:1, :]
    nbr = jnp.max(jnp.where(mg, c, NEG), axis=1, keepdims=True)
    nbr_out[...] = nbr + jnp.zeros((R, CH), jnp.float32)


def _deg_nbr(H, conf8):
    return pl.pallas_call(
        _deg_nbr_body,
        grid=(NB,),
        in_specs=[
            pl.BlockSpec((R, N), lambda i: (i, 0)),
            pl.BlockSpec((8, N), lambda i: (0, 0)),
        ],
        out_specs=[
            pl.BlockSpec((R, CH), lambda i: (i, 0)),
            pl.BlockSpec((R, CH), lambda i: (i, 0)),
        ],
        out_shape=[
            jax.ShapeDtypeStruct((N, CH), jnp.float32),
            jax.ShapeDtypeStruct((N, CH), jnp.float32),
        ],
        compiler_params=pltpu.CompilerParams(dimension_semantics=("parallel",)),
    )(H, conf8)


def _xyz_body(hb, d8, di, xyz_out):
    hblk = hb[...]
    mg = ((hblk + hblk) > 1.0).astype(jnp.float32)
    mvd = jnp.sum(mg * d8[...][0:1, :], axis=1, keepdims=True)
    drow = di[...][:, 0:1]
    xyz_out[...] = (drow * drow - mvd) + jnp.zeros((R, CH), jnp.float32)


def _xyz_k(H, d8, dcol):
    return pl.pallas_call(
        _xyz_body,
        grid=(NB,),
        in_specs=[
            pl.BlockSpec((R, N), lambda i: (i, 0)),
            pl.BlockSpec((8, N), lambda i: (0, 0)),
            pl.BlockSpec((R, CH), lambda i: (i, 0)),
        ],
        out_specs=pl.BlockSpec((R, CH), lambda i: (i, 0)),
        out_shape=jax.ShapeDtypeStruct((N, CH), jnp.float32),
        compiler_params=pltpu.CompilerParams(dimension_semantics=("parallel",)),
    )(H, d8, dcol)


_PROBES = 9


def _slot_of(occ, mask, h):
    js = jnp.arange(_PROBES + 1, dtype=jnp.int32)

    def cond_fn(st):
        return st[2] < 0

    def body_fn(st):
        i, perturb, _ = st
        valid = (js == 0) | (i + _PROBES <= mask)
        idxs = jnp.minimum(i + js, jnp.int32(occ.shape[0] - 1))
        hit = valid & jnp.logical_not(occ[idxs])
        jhit = jnp.min(jnp.where(hit, js, jnp.int32(_PROBES + 1)))
        found = jhit <= _PROBES
        slot = jnp.where(found, i + jhit, jnp.int32(-1))
        p2 = perturb >> 5
        i2 = (i * 5 + 1 + p2) & mask
        return (jnp.where(found, i, i2), jnp.where(found, perturb, p2), slot)

    st = jax.lax.while_loop(cond_fn, body_fn, (h & mask, h, jnp.int32(-1)))
    return st[2]


def _hset_add(keys, occ, mask, k):
    slot = _slot_of(occ, mask, k)
    return keys.at[slot].set(k), occ.at[slot].set(True)


def _hset_resize(keys, occ, newmask):
    def body(s, st):
        def ins(st_):
            return _hset_add(st_[0], st_[1], newmask, keys[s])

        return jax.lax.cond(occ[s], ins, lambda st_: st_, st)

    empty = (jnp.zeros_like(keys), jnp.zeros_like(occ))
    return jax.lax.fori_loop(0, occ.shape[0], body, empty)


def _hset_sim(elems, count):
    size = 512
    keys0 = jnp.zeros((size,), jnp.int32)
    occ0 = jnp.zeros((size,), bool)
    mask0 = jnp.int32(7)

    def body(t, st):
        def do(st_):
            keys, occ, mask = st_
            keys, occ = _hset_add(keys, occ, mask, elems[t])
            fill = t.astype(jnp.int32) + 1
            need = fill * 5 >= mask * 3
            newmask = jnp.where(mask == 7, jnp.int32(31),
                                jnp.where(mask == 31, jnp.int32(127),
                                          jnp.int32(511)))
            keys, occ = jax.lax.cond(
                need,
                lambda ko: _hset_resize(ko[0], ko[1], newmask),
                lambda ko: ko,
                (keys, occ))
            mask = jnp.where(need, newmask, mask)
            return keys, occ, mask

        return jax.lax.cond(t < count, do, lambda st_: st_, st)

    return jax.lax.fori_loop(0, elems.shape[0], body, (keys0, occ0, mask0))


def kernel(corr_pos, src_keypts, tgt_keypts, W_in, b_in, W_hid, b_hid,
           W_out, b_out, sigma):
    src = src_keypts[0]
    tgt = tgt_keypts[0]
    srcT = jnp.transpose(src)
    tgtT = jnp.transpose(tgt)

    fcg, sv = _fcg_sv(src, tgt, srcT, tgtT)
    thresh = sv.reshape(1, N, TOPK).reshape(1, -1).mean(axis=1)[:, None, None]
    thr = thresh.reshape(1, 1)

    H2d, nnz = _h_matmul(fcg, thr)
    flag = (nnz[0:1, 0:1] > 0).astype(jnp.float32)

    x = corr_pos[0]
    xp = jnp.concatenate([x, jnp.zeros((N, 2), jnp.float32)], axis=1)
    w_inp = jnp.concatenate([W_in, jnp.zeros((2, CH), jnp.float32)], axis=0)
    b_in8 = jnp.broadcast_to(b_in.reshape(1, CH), (8, CH))
    b_hid8 = jnp.broadcast_to(b_hid.reshape(1, CH), (8, CH))
    w_outp = jnp.concatenate(
        [W_out, jnp.zeros((CH, CH - 1), jnp.float32)], axis=1)
    bo = b_out.reshape(1, 1)

    esum, deg = _esum(flag, H2d, xp, w_inp, b_in8)
    msum = _msum(flag, H2d, esum, deg)
    h2, log128 = _mlp(bo, xp, w_inp, b_in8, msum, deg, W_hid, b_hid8, w_outp)

    logits = log128[:, 0:1]
    confidence = logits.reshape(1, -1)
    corr_feats = jnp.transpose(h2)[None, :, :]

    sig2 = (sigma ** 2).reshape(1, 1)
    M = _m_matmul(sig2, h2)[None]

    conf8 = jnp.broadcast_to(confidence, (8, N))
    d128, nbr128 = _deg_nbr(H2d, conf8)
    D = d128[:, 0].reshape(1, N)
    nbr_max = nbr128[:, 0].reshape(1, N)
    d8 = jnp.broadcast_to(D, (8, N))
    xyz128 = _xyz_k(H2d, d8, d128)
    xyz = xyz128[:, 0].reshape(1, 1, N)

    Lscore = jnp.linalg.norm(xyz, axis=1)
    low = Lscore.min(axis=1, keepdims=True)
    up = Lscore.max(axis=1, keepdims=True)
    Lscore = (Lscore - low) / (up - low) * (D > 0).astype(jnp.float32)
    ilm = jnp.where(D > 0,
                    (confidence >= nbr_max).astype(jnp.float32),
                    jnp.float32(jnp.inf))
    is_local_max = ilm * (D > 0).astype(jnp.float32)
    score = Lscore * is_local_max
    seed1 = jnp.argsort(-score, axis=1)
    seed2 = jnp.argsort(-Lscore, axis=1)
    max_num = int(N * 0.1)
    sel_len1 = jnp.minimum(jnp.int32(max_num),
                           (score > 0).sum().astype(jnp.int32))
    elems = seed1[0, :max_num].astype(jnp.int32)
    keys, occ, _ = _hset_sim(elems, sel_len1)
    order = jnp.cumsum(occ.astype(jnp.int32)) - 1
    set_list = jnp.zeros((max_num,), jnp.int32).at[
        jnp.where(occ, order, jnp.int32(max_num))].set(keys, mode='drop')
    valid = jnp.arange(max_num, dtype=jnp.int32) < sel_len1
    in_set = jnp.zeros((N,), bool).at[
        jnp.where(valid, elems, jnp.int32(N))].set(True, mode='drop')
    s2 = seed2[0].astype(jnp.int32)
    keep = jnp.logical_not(in_set[s2])
    rank = jnp.cumsum(keep.astype(jnp.int32)) - 1
    take = keep & (rank < max_num)
    uniq = jnp.zeros((max_num,), jnp.int32).at[
        jnp.where(take, rank, jnp.int32(max_num))].set(s2, mode='drop')
    j = jnp.arange(max_num, dtype=jnp.int32)
    appended = jnp.where(j < sel_len1, set_list[j],
                         uniq[jnp.clip(j - sel_len1, 0, max_num - 1)])
    seeds = appended[None, :]

    return confidence, corr_feats, M, H2d[None], seeds


# speedup vs baseline: 1.0148x; 1.0148x over previous
import functools

import jax
import jax.numpy as jnp
import numpy as np
from jax.experimental import pallas as pl
from jax.experimental.pallas import tpu as pltpu

N = 2000
R = 400
NB = N // R
CH = 128
TOPK = 200
SIG2 = np.float32(0.1 ** 2)
NEG = np.float32(-np.inf)


def _fcg_sv_body(src_r, tgt_r, srcT, tgtT, fcg_out, sv_out, vals):
    i = pl.program_id(0)

    def dist(own, allT):
        d2 = None
        for c in range(3):
            a = own[:, c:c + 1]
            b = allT[c:c + 1, :]
            e = a - b
            d2 = e * e if d2 is None else d2 + e * e
        return jnp.sqrt(jnp.maximum(d2, 0.0))

    pd = dist(src_r[...], srcT[...]) - dist(tgt_r[...], tgtT[...])
    fcg = jnp.maximum(1.0 - (pd * pd) / SIG2, 0.0)
    rows = i * R + jax.lax.broadcasted_iota(jnp.int32, (R, N), 0)
    cols = jax.lax.broadcasted_iota(jnp.int32, (R, N), 1)
    fcg = fcg * (1.0 - (rows == cols).astype(jnp.float32))
    fcg_out[...] = fcg
    vals[...] = fcg
    sv_out[...] = jnp.zeros((R, TOPK), jnp.float32)

    lane = jax.lax.broadcasted_iota(jnp.int32, (R, TOPK), 1)
    m0 = jnp.max(fcg, axis=1, keepdims=True)

    def body(it, carry):
        m, tpos = carry
        v = vals[...]
        eq = v == m
        w = jnp.where(eq, NEG, v)
        vals[...] = w
        cnt = jnp.sum(eq.astype(jnp.int32), axis=1, keepdims=True)
        mnext = jnp.max(w, axis=1, keepdims=True)
        mv = jnp.maximum(m, 0.0)
        win = (lane >= tpos) & (lane < tpos + cnt)
        sv_out[...] += jnp.where(win, mv, 0.0)
        return (mnext, tpos + cnt)

    jax.lax.fori_loop(0, TOPK, body,
                      (m0, jnp.zeros((R, 1), jnp.int32)))


def _fcg_sv(srcp, tgtp, srcT, tgtT):
    return pl.pallas_call(
        _fcg_sv_body,
        grid=(NB,),
        in_specs=[
            pl.BlockSpec((R, 3), lambda i: (i, 0)),
            pl.BlockSpec((R, 3), lambda i: (i, 0)),
            pl.BlockSpec((3, N), lambda i: (0, 0)),
            pl.BlockSpec((3, N), lambda i: (0, 0)),
        ],
        out_specs=[
            pl.BlockSpec((R, N), lambda i: (i, 0)),
            pl.BlockSpec((R, TOPK), lambda i: (i, 0)),
        ],
        out_shape=[
            jax.ShapeDtypeStruct((N, N), jnp.float32),
            jax.ShapeDtypeStruct((N, TOPK), jnp.float32),
        ],
        scratch_shapes=[pltpu.VMEM((R, N), jnp.float32)],
        compiler_params=pltpu.CompilerParams(dimension_semantics=("parallel",)),
    )(srcp, tgtp, srcT, tgtT)


def _h_body(thr, fa, fb, h_out, nnz_out):
    t = thr[0, 0]
    a = fa[...]
    a = jnp.where(a < t, 0.0, a)
    b = fb[...]
    b = jnp.where(b < t, 0.0, b)
    h = jnp.dot(a, b, preferred_element_type=jnp.float32) * a
    h_out[...] = h
    cnt = jnp.sum((h > 0).astype(jnp.float32))

    @pl.when(pl.program_id(0) == 0)
    def _():
        nnz_out[...] = jnp.zeros_like(nnz_out)

    nnz_out[...] += cnt


def _h_matmul(fcg, thresh):
    return pl.pallas_call(
        _h_body,
        grid=(NB,),
        in_specs=[
            pl.BlockSpec(memory_space=pltpu.SMEM),
            pl.BlockSpec((R, N), lambda i: (i, 0)),
            pl.BlockSpec((N, N), lambda i: (0, 0)),
        ],
        out_specs=[
            pl.BlockSpec((R, N), lambda i: (i, 0)),
            pl.BlockSpec((8, 128), lambda i: (0, 0)),
        ],
        out_shape=[
            jax.ShapeDtypeStruct((N, N), jnp.float32),
            jax.ShapeDtypeStruct((8, 128), jnp.float32),
        ],
    )(thresh, fcg, fcg)


def _adj_row(hblk, flag, i):
    a = (hblk > 0).astype(jnp.float32)
    rows = i * R + jax.lax.broadcasted_iota(jnp.int32, (R, N), 0)
    cols = jax.lax.broadcasted_iota(jnp.int32, (R, N), 1)
    eye = (rows == cols).astype(jnp.float32)
    return jnp.where(flag > 0, a, eye)


def _esum_body(flag, hb, xfull, w_in, b_in, esum_out, deg_out):
    i = pl.program_id(0)
    a = _adj_row(hb[...], flag[0, 0], i)
    h = jnp.maximum(jnp.dot(xfull[...], w_in[...],
                            preferred_element_type=jnp.float32)
                    + b_in[0:1, :], 0.0)
    esum_out[...] = jnp.dot(a, h, preferred_element_type=jnp.float32)
    deg_out[...] = jnp.sum(a, axis=1, keepdims=True) + jnp.zeros((R, CH), jnp.float32)


def _esum(flag, H, xp, w_inp, b_in8):
    return pl.pallas_call(
        _esum_body,
        grid=(NB,),
        in_specs=[
            pl.BlockSpec(memory_space=pltpu.SMEM),
            pl.BlockSpec((R, N), lambda i: (i, 0)),
            pl.BlockSpec((N, 8), lambda i: (0, 0)),
            pl.BlockSpec((8, CH), lambda i: (0, 0)),
            pl.BlockSpec((8, CH), lambda i: (0, 0)),
        ],
        out_specs=[
            pl.BlockSpec((R, CH), lambda i: (i, 0)),
            pl.BlockSpec((R, CH), lambda i: (i, 0)),
        ],
        out_shape=[
            jax.ShapeDtypeStruct((N, CH), jnp.float32),
            jax.ShapeDtypeStruct((N, CH), jnp.float32),
        ],
        compiler_params=pltpu.CompilerParams(dimension_semantics=("parallel",)),
    )(flag, H, xp, w_inp, b_in8)


def _msum_body(flag, hb, esum, deg, msum_out):
    i = pl.program_id(0)
    a = _adj_row(hb[...], flag[0, 0], i)
    emean = esum[...] / jnp.maximum(deg[...][:, 0:1], 1.0)
    msum_out[...] = jnp.dot(a, emean, preferred_element_type=jnp.float32)


def _msum(flag, H, esum, deg):
    return pl.pallas_call(
        _msum_body,
        grid=(NB,),
        in_specs=[
            pl.BlockSpec(memory_space=pltpu.SMEM),
            pl.BlockSpec((R, N), lambda i: (i, 0)),
            pl.BlockSpec((N, CH), lambda i: (0, 0)),
            pl.BlockSpec((N, CH), lambda i: (0, 0)),
        ],
        out_specs=pl.BlockSpec((R, CH), lambda i: (i, 0)),
        out_shape=jax.ShapeDtypeStruct((N, CH), jnp.float32),
        compiler_params=pltpu.CompilerParams(dimension_semantics=("parallel",)),
    )(flag, H, esum, deg)


def _mlp_body(bo, xi, w_in, b_in, msumi, degi, w_hid, b_hid, w_outp,
              h2_out, log_out):
    hk = jnp.maximum(jnp.dot(xi[...], w_in[...],
                             preferred_element_type=jnp.float32)
                     + b_in[0:1, :], 0.0)
    m = msumi[...] / jnp.maximum(degi[...][:, 0:1], 1.0)
    h2 = jnp.maximum(jnp.dot(hk + m, w_hid[...],
                             preferred_element_type=jnp.float32)
                     + b_hid[0:1, :], 0.0)
    h2_out[...] = h2
    log_out[...] = jnp.dot(h2, w_outp[...],
                           preferred_element_type=jnp.float32) + bo[0, 0]


def _mlp(b_out, xp, w_inp, b_in8, msum, deg, w_hid, b_hid8, w_outp):
    return pl.pallas_call(
        _mlp_body,
        grid=(NB,),
        in_specs=[
            pl.BlockSpec(memory_space=pltpu.SMEM),
            pl.BlockSpec((R, 8), lambda i: (i, 0)),
            pl.BlockSpec((8, CH), lambda i: (0, 0)),
            pl.BlockSpec((8, CH), lambda i: (0, 0)),
            pl.BlockSpec((R, CH), lambda i: (i, 0)),
            pl.BlockSpec((R, CH), lambda i: (i, 0)),
            pl.BlockSpec((CH, CH), lambda i: (0, 0)),
            pl.BlockSpec((8, CH), lambda i: (0, 0)),
            pl.BlockSpec((CH, CH), lambda i: (0, 0)),
        ],
        out_specs=[
            pl.BlockSpec((R, CH), lambda i: (i, 0)),
            pl.BlockSpec((R, CH), lambda i: (i, 0)),
        ],
        out_shape=[
            jax.ShapeDtypeStruct((N, CH), jnp.float32),
            jax.ShapeDtypeStruct((N, CH), jnp.float32),
        ],
        compiler_params=pltpu.CompilerParams(dimension_semantics=("parallel",)),
    )(b_out, xp, w_inp, b_in8, msum, deg, w_hid, b_hid8, w_outp)


def _m_body(s2, ai, bfull, m_out):
    i = pl.program_id(0)
    acc = jax.lax.dot_general(ai[...], bfull[...], (((1,), (1,)), ((), ())),
                              preferred_element_type=jnp.float32)
    v = jnp.clip(1.0 - (1.0 - acc) / s2[0, 0], 0.0, 1.0)
    rows = i * R + jax.lax.broadcasted_iota(jnp.int32, (R, N), 0)
    cols = jax.lax.broadcasted_iota(jnp.int32, (R, N), 1)
    m_out[...] = v * (1.0 - (rows == cols).astype(jnp.float32))


def _m_matmul(sig2, h2):
    return pl.pallas_call(
        _m_body,
        grid=(NB,),
        in_specs=[
            pl.BlockSpec(memory_space=pltpu.SMEM),
            pl.BlockSpec((R, CH), lambda i: (i, 0)),
            pl.BlockSpec((N, CH), lambda i: (0, 0)),
        ],
        out_specs=pl.BlockSpec((R, N), lambda i: (i, 0)),
        out_shape=jax.ShapeDtypeStruct((N, N), jnp.float32),
        compiler_params=pltpu.CompilerParams(dimension_semantics=("parallel",)),
    )(sig2, h2, h2)


def _deg_nbr_body(hb, conf8, d_out, nbr_out):
    hblk = hb[...]
    mg = (hblk + hblk) > 1.0
    d = jnp.sum(mg.astype(jnp.float32), axis=1, keepdims=True)
    d_out[...] = d + jnp.zeros((R, CH), jnp.float32)
    c = conf8[...][0:1, :]
    nbr = jnp.max(jnp.where(mg, c, NEG), axis=1, keepdims=True)
    nbr_out[...] = nbr + jnp.zeros((R, CH), jnp.float32)


def _deg_nbr(H, conf8):
    return pl.pallas_call(
        _deg_nbr_body,
        grid=(NB,),
        in_specs=[
            pl.BlockSpec((R, N), lambda i: (i, 0)),
            pl.BlockSpec((8, N), lambda i: (0, 0)),
        ],
        out_specs=[
            pl.BlockSpec((R, CH), lambda i: (i, 0)),
            pl.BlockSpec((R, CH), lambda i: (i, 0)),
        ],
        out_shape=[
            jax.ShapeDtypeStruct((N, CH), jnp.float32),
            jax.ShapeDtypeStruct((N, CH), jnp.float32),
        ],
        compiler_params=pltpu.CompilerParams(dimension_semantics=("parallel",)),
    )(H, conf8)


def _xyz_body(hb, d8, di, xyz_out):
    hblk = hb[...]
    mg = ((hblk + hblk) > 1.0).astype(jnp.float32)
    mvd = jnp.sum(mg * d8[...][0:1, :], axis=1, keepdims=True)
    drow = di[...][:, 0:1]
    xyz_out[...] = (drow * drow - mvd) + jnp.zeros((R, CH), jnp.float32)


def _xyz_k(H, d8, dcol):
    return pl.pallas_call(
        _xyz_body,
        grid=(NB,),
        in_specs=[
            pl.BlockSpec((R, N), lambda i: (i, 0)),
            pl.BlockSpec((8, N), lambda i: (0, 0)),
            pl.BlockSpec((R, CH), lambda i: (i, 0)),
        ],
        out_specs=pl.BlockSpec((R, CH), lambda i: (i, 0)),
        out_shape=jax.ShapeDtypeStruct((N, CH), jnp.float32),
        compiler_params=pltpu.CompilerParams(dimension_semantics=("parallel",)),
    )(H, d8, dcol)


_PROBES = 9


def _slot_of(occ, mask, h):
    js = jnp.arange(_PROBES + 1, dtype=jnp.int32)

    def cond_fn(st):
        return st[2] < 0

    def body_fn(st):
        i, perturb, _ = st
        valid = (js == 0) | (i + _PROBES <= mask)
        idxs = jnp.minimum(i + js, jnp.int32(occ.shape[0] - 1))
        hit = valid & jnp.logical_not(occ[idxs])
        jhit = jnp.min(jnp.where(hit, js, jnp.int32(_PROBES + 1)))
        found = jhit <= _PROBES
        slot = jnp.where(found, i + jhit, jnp.int32(-1))
        p2 = perturb >> 5
        i2 = (i * 5 + 1 + p2) & mask
        return (jnp.where(found, i, i2), jnp.where(found, perturb, p2), slot)

    st = jax.lax.while_loop(cond_fn, body_fn, (h & mask, h, jnp.int32(-1)))
    return st[2]


def _hset_add(keys, occ, mask, k):
    slot = _slot_of(occ, mask, k)
    return keys.at[slot].set(k), occ.at[slot].set(True)


def _hset_resize(keys, occ, newmask):
    def body(s, st):
        def ins(st_):
            return _hset_add(st_[0], st_[1], newmask, keys[s])

        return jax.lax.cond(occ[s], ins, lambda st_: st_, st)

    empty = (jnp.zeros_like(keys), jnp.zeros_like(occ))
    return jax.lax.fori_loop(0, occ.shape[0], body, empty)


def _hset_sim(elems, count):
    size = 512
    keys0 = jnp.zeros((size,), jnp.int32)
    occ0 = jnp.zeros((size,), bool)
    mask0 = jnp.int32(7)

    def body(t, st):
        def do(st_):
            keys, occ, mask = st_
            keys, occ = _hset_add(keys, occ, mask, elems[t])
            fill = t.astype(jnp.int32) + 1
            need = fill * 5 >= mask * 3
            newmask = jnp.where(mask == 7, jnp.int32(31),
                                jnp.where(mask == 31, jnp.int32(127),
                                          jnp.int32(511)))
            keys, occ = jax.lax.cond(
                need,
                lambda ko: _hset_resize(ko[0], ko[1], newmask),
                lambda ko: ko,
                (keys, occ))
            mask = jnp.where(need, newmask, mask)
            return keys, occ, mask

        return jax.lax.cond(t < count, do, lambda st_: st_, st)

    return jax.lax.fori_loop(0, elems.shape[0], body, (keys0, occ0, mask0))


def kernel(corr_pos, src_keypts, tgt_keypts, W_in, b_in, W_hid, b_hid,
           W_out, b_out, sigma):
    src = src_keypts[0]
    tgt = tgt_keypts[0]
    srcT = jnp.transpose(src)
    tgtT = jnp.transpose(tgt)

    fcg, sv = _fcg_sv(src, tgt, srcT, tgtT)
    thresh = sv.reshape(1, N, TOPK).reshape(1, -1).mean(axis=1)[:, None, None]
    thr = thresh.reshape(1, 1)

    H2d, nnz = _h_matmul(fcg, thr)
    flag = (nnz[0:1, 0:1] > 0).astype(jnp.float32)

    x = corr_pos[0]
    xp = jnp.concatenate([x, jnp.zeros((N, 2), jnp.float32)], axis=1)
    w_inp = jnp.concatenate([W_in, jnp.zeros((2, CH), jnp.float32)], axis=0)
    b_in8 = jnp.broadcast_to(b_in.reshape(1, CH), (8, CH))
    b_hid8 = jnp.broadcast_to(b_hid.reshape(1, CH), (8, CH))
    w_outp = jnp.concatenate(
        [W_out, jnp.zeros((CH, CH - 1), jnp.float32)], axis=1)
    bo = b_out.reshape(1, 1)

    esum, deg = _esum(flag, H2d, xp, w_inp, b_in8)
    msum = _msum(flag, H2d, esum, deg)
    h2, log128 = _mlp(bo, xp, w_inp, b_in8, msum, deg, W_hid, b_hid8, w_outp)

    logits = log128[:, 0:1]
    confidence = logits.reshape(1, -1)
    corr_feats = jnp.transpose(h2)[None, :, :]

    sig2 = (sigma ** 2).reshape(1, 1)
    M = _m_matmul(sig2, h2)[None]

    conf8 = jnp.broadcast_to(confidence, (8, N))
    d128, nbr128 = _deg_nbr(H2d, conf8)
    D = d128[:, 0].reshape(1, N)
    nbr_max = nbr128[:, 0].reshape(1, N)
    d8 = jnp.broadcast_to(D, (8, N))
    xyz128 = _xyz_k(H2d, d8, d128)
    xyz = xyz128[:, 0].reshape(1, 1, N)

    Lscore = jnp.linalg.norm(xyz, axis=1)
    low = Lscore.min(axis=1, keepdims=True)
    up = Lscore.max(axis=1, keepdims=True)
    Lscore = (Lscore - low) / (up - low) * (D > 0).astype(jnp.float32)
    ilm = jnp.where(D > 0,
                    (confidence >= nbr_max).astype(jnp.float32),
                    jnp.float32(jnp.inf))
    is_local_max = ilm * (D > 0).astype(jnp.float32)
    score = Lscore * is_local_max
    seed1 = jnp.argsort(-score, axis=1)
    seed2 = jnp.argsort(-Lscore, axis=1)
    max_num = int(N * 0.1)
    sel_len1 = jnp.minimum(jnp.int32(max_num),
                           (score > 0).sum().astype(jnp.int32))
    elems = seed1[0, :max_num].astype(jnp.int32)
    keys, occ, _ = _hset_sim(elems, sel_len1)
    order = jnp.cumsum(occ.astype(jnp.int32)) - 1
    set_list = jnp.zeros((max_num,), jnp.int32).at[
        jnp.where(occ, order, jnp.int32(max_num))].set(keys, mode='drop')
    valid = jnp.arange(max_num, dtype=jnp.int32) < sel_len1
    in_set = jnp.zeros((N,), bool).at[
        jnp.where(valid, elems, jnp.int32(N))].set(True, mode='drop')
    s2 = seed2[0].astype(jnp.int32)
    keep = jnp.logical_not(in_set[s2])
    rank = jnp.cumsum(keep.astype(jnp.int32)) - 1
    take = keep & (rank < max_num)
    uniq = jnp.zeros((max_num,), jnp.int32).at[
        jnp.where(take, rank, jnp.int32(max_num))].set(s2, mode='drop')
    j = jnp.arange(max_num, dtype=jnp.int32)
    appended = jnp.where(j < sel_len1, set_list[j],
                         uniq[jnp.clip(j - sel_len1, 0, max_num - 1)])
    seeds = appended[None, :]

    return confidence, corr_feats, M, H2d[None], seeds


# speedup vs baseline: 1.1269x; 1.1104x over previous
import functools

import jax
import jax.numpy as jnp
import numpy as np
from jax.experimental import pallas as pl
from jax.experimental.pallas import tpu as pltpu

N = 2000
R = 400
NB = N // R
CH = 128
TOPK = 200
SIG2 = np.float32(0.1 ** 2)
NEG = np.float32(-np.inf)


def _fcg_sv_body(src_r, tgt_r, srcT, tgtT, fcg_out, sv_out, vals):
    i = pl.program_id(0)

    def dist(own, allT):
        d2 = None
        for c in range(3):
            a = own[:, c:c + 1]
            b = allT[c:c + 1, :]
            e = a - b
            d2 = e * e if d2 is None else d2 + e * e
        return jnp.sqrt(jnp.maximum(d2, 0.0))

    pd = dist(src_r[...], srcT[...]) - dist(tgt_r[...], tgtT[...])
    fcg = jnp.maximum(1.0 - (pd * pd) / SIG2, 0.0)
    rows = i * R + jax.lax.broadcasted_iota(jnp.int32, (R, N), 0)
    cols = jax.lax.broadcasted_iota(jnp.int32, (R, N), 1)
    fcg = fcg * (1.0 - (rows == cols).astype(jnp.float32))
    fcg_out[...] = fcg
    vals[...] = fcg
    sv_out[...] = jnp.zeros((R, TOPK), jnp.float32)

    lane = jax.lax.broadcasted_iota(jnp.int32, (R, TOPK), 1)
    m0 = jnp.max(fcg, axis=1, keepdims=True)

    def body(it, carry):
        m, tpos = carry
        v = vals[...]
        eq = v == m
        w = jnp.where(eq, NEG, v)
        vals[...] = w
        cnt = jnp.sum(eq.astype(jnp.int32), axis=1, keepdims=True)
        mnext = jnp.max(w, axis=1, keepdims=True)
        mv = jnp.maximum(m, 0.0)
        win = (lane >= tpos) & (lane < tpos + cnt)
        sv_out[...] += jnp.where(win, mv, 0.0)
        return (mnext, tpos + cnt)

    jax.lax.fori_loop(0, TOPK, body,
                      (m0, jnp.zeros((R, 1), jnp.int32)))


def _fcg_sv(srcp, tgtp, srcT, tgtT):
    return pl.pallas_call(
        _fcg_sv_body,
        grid=(NB,),
        in_specs=[
            pl.BlockSpec((R, 3), lambda i: (i, 0)),
            pl.BlockSpec((R, 3), lambda i: (i, 0)),
            pl.BlockSpec((3, N), lambda i: (0, 0)),
            pl.BlockSpec((3, N), lambda i: (0, 0)),
        ],
        out_specs=[
            pl.BlockSpec((R, N), lambda i: (i, 0)),
            pl.BlockSpec((R, TOPK), lambda i: (i, 0)),
        ],
        out_shape=[
            jax.ShapeDtypeStruct((N, N), jnp.float32),
            jax.ShapeDtypeStruct((N, TOPK), jnp.float32),
        ],
        scratch_shapes=[pltpu.VMEM((R, N), jnp.float32)],
        compiler_params=pltpu.CompilerParams(dimension_semantics=("parallel",)),
    )(srcp, tgtp, srcT, tgtT)


def _h_body(thr, fa, fb, h_out, nnz_out):
    t = thr[0, 0]
    a = fa[...]
    a = jnp.where(a < t, 0.0, a)
    b = fb[...]
    b = jnp.where(b < t, 0.0, b)
    h = jnp.dot(a, b, preferred_element_type=jnp.float32) * a
    h_out[...] = h
    cnt = jnp.sum((h > 0).astype(jnp.float32))

    @pl.when(pl.program_id(0) == 0)
    def _():
        nnz_out[...] = jnp.zeros_like(nnz_out)

    nnz_out[...] += cnt


def _h_matmul(fcg, thresh):
    return pl.pallas_call(
        _h_body,
        grid=(NB,),
        in_specs=[
            pl.BlockSpec(memory_space=pltpu.SMEM),
            pl.BlockSpec((R, N), lambda i: (i, 0)),
            pl.BlockSpec((N, N), lambda i: (0, 0)),
        ],
        out_specs=[
            pl.BlockSpec((R, N), lambda i: (i, 0)),
            pl.BlockSpec((8, 128), lambda i: (0, 0)),
        ],
        out_shape=[
            jax.ShapeDtypeStruct((N, N), jnp.float32),
            jax.ShapeDtypeStruct((8, 128), jnp.float32),
        ],
    )(thresh, fcg, fcg)


def _adj_row(hblk, flag, i):
    a = (hblk > 0).astype(jnp.float32)
    rows = i * R + jax.lax.broadcasted_iota(jnp.int32, (R, N), 0)
    cols = jax.lax.broadcasted_iota(jnp.int32, (R, N), 1)
    eye = (rows == cols).astype(jnp.float32)
    return jnp.where(flag > 0, a, eye)


def _esum_body(flag, hb, xfull, w_in, b_in, esum_out, deg_out):
    i = pl.program_id(0)
    a = _adj_row(hb[...], flag[0, 0], i)
    h = jnp.maximum(jnp.dot(xfull[...], w_in[...],
                            preferred_element_type=jnp.float32)
                    + b_in[0:1, :], 0.0)
    esum_out[...] = jnp.dot(a, h, preferred_element_type=jnp.float32)
    deg_out[...] = jnp.sum(a, axis=1, keepdims=True) + jnp.zeros((R, CH), jnp.float32)


def _esum(flag, H, xp, w_inp, b_in8):
    return pl.pallas_call(
        _esum_body,
        grid=(NB,),
        in_specs=[
            pl.BlockSpec(memory_space=pltpu.SMEM),
            pl.BlockSpec((R, N), lambda i: (i, 0)),
            pl.BlockSpec((N, 8), lambda i: (0, 0)),
            pl.BlockSpec((8, CH), lambda i: (0, 0)),
            pl.BlockSpec((8, CH), lambda i: (0, 0)),
        ],
        out_specs=[
            pl.BlockSpec((R, CH), lambda i: (i, 0)),
            pl.BlockSpec((R, CH), lambda i: (i, 0)),
        ],
        out_shape=[
            jax.ShapeDtypeStruct((N, CH), jnp.float32),
            jax.ShapeDtypeStruct((N, CH), jnp.float32),
        ],
        compiler_params=pltpu.CompilerParams(dimension_semantics=("parallel",)),
    )(flag, H, xp, w_inp, b_in8)


def _msum_body(flag, hb, esum, deg, msum_out):
    i = pl.program_id(0)
    a = _adj_row(hb[...], flag[0, 0], i)
    emean = esum[...] / jnp.maximum(deg[...][:, 0:1], 1.0)
    msum_out[...] = jnp.dot(a, emean, preferred_element_type=jnp.float32)


def _msum(flag, H, esum, deg):
    return pl.pallas_call(
        _msum_body,
        grid=(NB,),
        in_specs=[
            pl.BlockSpec(memory_space=pltpu.SMEM),
            pl.BlockSpec((R, N), lambda i: (i, 0)),
            pl.BlockSpec((N, CH), lambda i: (0, 0)),
            pl.BlockSpec((N, CH), lambda i: (0, 0)),
        ],
        out_specs=pl.BlockSpec((R, CH), lambda i: (i, 0)),
        out_shape=jax.ShapeDtypeStruct((N, CH), jnp.float32),
        compiler_params=pltpu.CompilerParams(dimension_semantics=("parallel",)),
    )(flag, H, esum, deg)


def _mlp_body(bo, xi, w_in, b_in, msumi, degi, w_hid, b_hid, w_outp,
              h2_out, log_out):
    hk = jnp.maximum(jnp.dot(xi[...], w_in[...],
                             preferred_element_type=jnp.float32)
                     + b_in[0:1, :], 0.0)
    m = msumi[...] / jnp.maximum(degi[...][:, 0:1], 1.0)
    h2 = jnp.maximum(jnp.dot(hk + m, w_hid[...],
                             preferred_element_type=jnp.float32)
                     + b_hid[0:1, :], 0.0)
    h2_out[...] = h2
    log_out[...] = jnp.dot(h2, w_outp[...],
                           preferred_element_type=jnp.float32) + bo[0, 0]


def _mlp(b_out, xp, w_inp, b_in8, msum, deg, w_hid, b_hid8, w_outp):
    return pl.pallas_call(
        _mlp_body,
        grid=(NB,),
        in_specs=[
            pl.BlockSpec(memory_space=pltpu.SMEM),
            pl.BlockSpec((R, 8), lambda i: (i, 0)),
            pl.BlockSpec((8, CH), lambda i: (0, 0)),
            pl.BlockSpec((8, CH), lambda i: (0, 0)),
            pl.BlockSpec((R, CH), lambda i: (i, 0)),
            pl.BlockSpec((R, CH), lambda i: (i, 0)),
            pl.BlockSpec((CH, CH), lambda i: (0, 0)),
            pl.BlockSpec((8, CH), lambda i: (0, 0)),
            pl.BlockSpec((CH, CH), lambda i: (0, 0)),
        ],
        out_specs=[
            pl.BlockSpec((R, CH), lambda i: (i, 0)),
            pl.BlockSpec((R, CH), lambda i: (i, 0)),
        ],
        out_shape=[
            jax.ShapeDtypeStruct((N, CH), jnp.float32),
            jax.ShapeDtypeStruct((N, CH), jnp.float32),
        ],
        compiler_params=pltpu.CompilerParams(dimension_semantics=("parallel",)),
    )(b_out, xp, w_inp, b_in8, msum, deg, w_hid, b_hid8, w_outp)


def _m_body(s2, ai, bfull, m_out):
    i = pl.program_id(0)
    acc = jax.lax.dot_general(ai[...], bfull[...], (((1,), (1,)), ((), ())),
                              preferred_element_type=jnp.float32)
    v = jnp.clip(1.0 - (1.0 - acc) / s2[0, 0], 0.0, 1.0)
    rows = i * R + jax.lax.broadcasted_iota(jnp.int32, (R, N), 0)
    cols = jax.lax.broadcasted_iota(jnp.int32, (R, N), 1)
    m_out[...] = v * (1.0 - (rows == cols).astype(jnp.float32))


def _m_matmul(sig2, h2):
    return pl.pallas_call(
        _m_body,
        grid=(NB,),
        in_specs=[
            pl.BlockSpec(memory_space=pltpu.SMEM),
            pl.BlockSpec((R, CH), lambda i: (i, 0)),
            pl.BlockSpec((N, CH), lambda i: (0, 0)),
        ],
        out_specs=pl.BlockSpec((R, N), lambda i: (i, 0)),
        out_shape=jax.ShapeDtypeStruct((N, N), jnp.float32),
        compiler_params=pltpu.CompilerParams(dimension_semantics=("parallel",)),
    )(sig2, h2, h2)


def _deg_nbr_body(hb, conf8, d_out, nbr_out):
    hblk = hb[...]
    mg = (hblk + hblk) > 1.0
    d = jnp.sum(mg.astype(jnp.float32), axis=1, keepdims=True)
    d_out[...] = d + jnp.zeros((R, CH), jnp.float32)
    c = conf8[...][0:1, :]
    nbr = jnp.max(jnp.where(mg, c, NEG), axis=1, keepdims=True)
    nbr_out[...] = nbr + jnp.zeros((R, CH), jnp.float32)


def _deg_nbr(H, conf8):
    return pl.pallas_call(
        _deg_nbr_body,
        grid=(NB,),
        in_specs=[
            pl.BlockSpec((R, N), lambda i: (i, 0)),
            pl.BlockSpec((8, N), lambda i: (0, 0)),
        ],
        out_specs=[
            pl.BlockSpec((R, CH), lambda i: (i, 0)),
            pl.BlockSpec((R, CH), lambda i: (i, 0)),
        ],
        out_shape=[
            jax.ShapeDtypeStruct((N, CH), jnp.float32),
            jax.ShapeDtypeStruct((N, CH), jnp.float32),
        ],
        compiler_params=pltpu.CompilerParams(dimension_semantics=("parallel",)),
    )(H, conf8)


def _xyz_body(hb, d8, di, xyz_out):
    hblk = hb[...]
    mg = ((hblk + hblk) > 1.0).astype(jnp.float32)
    mvd = jnp.sum(mg * d8[...][0:1, :], axis=1, keepdims=True)
    drow = di[...][:, 0:1]
    xyz_out[...] = (drow * drow - mvd) + jnp.zeros((R, CH), jnp.float32)


def _xyz_k(H, d8, dcol):
    return pl.pallas_call(
        _xyz_body,
        grid=(NB,),
        in_specs=[
            pl.BlockSpec((R, N), lambda i: (i, 0)),
            pl.BlockSpec((8, N), lambda i: (0, 0)),
            pl.BlockSpec((R, CH), lambda i: (i, 0)),
        ],
        out_specs=pl.BlockSpec((R, CH), lambda i: (i, 0)),
        out_shape=jax.ShapeDtypeStruct((N, CH), jnp.float32),
        compiler_params=pltpu.CompilerParams(dimension_semantics=("parallel",)),
    )(H, d8, dcol)


_PROBES = 9


def _slot_of(occ, mask, h):
    js = jnp.arange(_PROBES + 1, dtype=jnp.int32)

    def cond_fn(st):
        return st[2] < 0

    def body_fn(st):
        i, perturb, _ = st
        valid = (js == 0) | (i + _PROBES <= mask)
        idxs = jnp.minimum(i + js, jnp.int32(occ.shape[0] - 1))
        hit = valid & jnp.logical_not(occ[idxs])
        jhit = jnp.min(jnp.where(hit, js, jnp.int32(_PROBES + 1)))
        found = jhit <= _PROBES
        slot = jnp.where(found, i + jhit, jnp.int32(-1))
        p2 = perturb >> 5
        i2 = (i * 5 + 1 + p2) & mask
        return (jnp.where(found, i, i2), jnp.where(found, perturb, p2), slot)

    st = jax.lax.while_loop(cond_fn, body_fn, (h & mask, h, jnp.int32(-1)))
    return st[2]


def _hset_add(keys, occ, mask, k):
    slot = _slot_of(occ, mask, k)
    return keys.at[slot].set(k), occ.at[slot].set(True)


def _hset_resize(keys, occ, newmask):
    def body(s, st):
        def ins(st_):
            return _hset_add(st_[0], st_[1], newmask, keys[s])

        return jax.lax.cond(occ[s], ins, lambda st_: st_, st)

    empty = (jnp.zeros_like(keys), jnp.zeros_like(occ))
    return jax.lax.fori_loop(0, occ.shape[0], body, empty)


def _hset_sim(elems, count):
    size = 512
    keys = jnp.zeros((size,), jnp.int32)
    occ = jnp.zeros((size,), bool)

    def seg(keys, occ, mask, lo, hi):
        def body(t, ko):
            k, o = ko
            slot = _slot_of(o, jnp.int32(mask), elems[t])
            slot = jnp.where(t < count, slot, jnp.int32(size))
            return (k.at[slot].set(elems[t], mode='drop'),
                    o.at[slot].set(True, mode='drop'))

        return jax.lax.fori_loop(lo, hi, body, (keys, occ))

    def resize(keys, occ, newmask, nkeys):
        order = jnp.cumsum(occ.astype(jnp.int32)) - 1
        compact = jnp.zeros((size,), jnp.int32).at[
            jnp.where(occ, order, jnp.int32(size))].set(keys, mode='drop')

        def body(s, ko):
            return _hset_add(ko[0], ko[1], jnp.int32(newmask), compact[s])

        empty = (jnp.zeros_like(keys), jnp.zeros_like(occ))
        return jax.lax.fori_loop(0, nkeys, body, empty)

    keys, occ = seg(keys, occ, 7, 0, 5)
    keys, occ = jax.lax.cond(count > 4,
                             lambda ko: resize(ko[0], ko[1], 31, 5),
                             lambda ko: ko, (keys, occ))
    keys, occ = seg(keys, occ, 31, 5, 19)
    keys, occ = jax.lax.cond(count > 18,
                             lambda ko: resize(ko[0], ko[1], 127, 19),
                             lambda ko: ko, (keys, occ))
    keys, occ = seg(keys, occ, 127, 19, 77)
    keys, occ = jax.lax.cond(count > 76,
                             lambda ko: resize(ko[0], ko[1], 511, 77),
                             lambda ko: ko, (keys, occ))
    keys, occ = seg(keys, occ, 511, 77, elems.shape[0])
    return keys, occ, jnp.int32(511)


def kernel(corr_pos, src_keypts, tgt_keypts, W_in, b_in, W_hid, b_hid,
           W_out, b_out, sigma):
    src = src_keypts[0]
    tgt = tgt_keypts[0]
    srcT = jnp.transpose(src)
    tgtT = jnp.transpose(tgt)

    fcg, sv = _fcg_sv(src, tgt, srcT, tgtT)
    thresh = sv.reshape(1, N, TOPK).reshape(1, -1).mean(axis=1)[:, None, None]
    thr = thresh.reshape(1, 1)

    H2d, nnz = _h_matmul(fcg, thr)
    flag = (nnz[0:1, 0:1] > 0).astype(jnp.float32)

    x = corr_pos[0]
    xp = jnp.concatenate([x, jnp.zeros((N, 2), jnp.float32)], axis=1)
    w_inp = jnp.concatenate([W_in, jnp.zeros((2, CH), jnp.float32)], axis=0)
    b_in8 = jnp.broadcast_to(b_in.reshape(1, CH), (8, CH))
    b_hid8 = jnp.broadcast_to(b_hid.reshape(1, CH), (8, CH))
    w_outp = jnp.concatenate(
        [W_out, jnp.zeros((CH, CH - 1), jnp.float32)], axis=1)
    bo = b_out.reshape(1, 1)

    esum, deg = _esum(flag, H2d, xp, w_inp, b_in8)
    msum = _msum(flag, H2d, esum, deg)
    h2, log128 = _mlp(bo, xp, w_inp, b_in8, msum, deg, W_hid, b_hid8, w_outp)

    logits = log128[:, 0:1]
    confidence = logits.reshape(1, -1)
    corr_feats = jnp.transpose(h2)[None, :, :]

    sig2 = (sigma ** 2).reshape(1, 1)
    M = _m_matmul(sig2, h2)[None]

    conf8 = jnp.broadcast_to(confidence, (8, N))
    d128, nbr128 = _deg_nbr(H2d, conf8)
    D = d128[:, 0].reshape(1, N)
    nbr_max = nbr128[:, 0].reshape(1, N)
    d8 = jnp.broadcast_to(D, (8, N))
    xyz128 = _xyz_k(H2d, d8, d128)
    xyz = xyz128[:, 0].reshape(1, 1, N)

    Lscore = jnp.linalg.norm(xyz, axis=1)
    low = Lscore.min(axis=1, keepdims=True)
    up = Lscore.max(axis=1, keepdims=True)
    Lscore = (Lscore - low) / (up - low) * (D > 0).astype(jnp.float32)
    ilm = jnp.where(D > 0,
                    (confidence >= nbr_max).astype(jnp.float32),
                    jnp.float32(jnp.inf))
    is_local_max = ilm * (D > 0).astype(jnp.float32)
    score = Lscore * is_local_max
    seed1 = jnp.argsort(-score, axis=1)
    seed2 = jnp.argsort(-Lscore, axis=1)
    max_num = int(N * 0.1)
    sel_len1 = jnp.minimum(jnp.int32(max_num),
                           (score > 0).sum().astype(jnp.int32))
    elems = seed1[0, :max_num].astype(jnp.int32)
    keys, occ, _ = _hset_sim(elems, sel_len1)
    order = jnp.cumsum(occ.astype(jnp.int32)) - 1
    set_list = jnp.zeros((max_num,), jnp.int32).at[
        jnp.where(occ, order, jnp.int32(max_num))].set(keys, mode='drop')
    valid = jnp.arange(max_num, dtype=jnp.int32) < sel_len1
    in_set = jnp.zeros((N,), bool).at[
        jnp.where(valid, elems, jnp.int32(N))].set(True, mode='drop')
    s2 = seed2[0].astype(jnp.int32)
    keep = jnp.logical_not(in_set[s2])
    rank = jnp.cumsum(keep.astype(jnp.int32)) - 1
    take = keep & (rank < max_num)
    uniq = jnp.zeros((max_num,), jnp.int32).at[
        jnp.where(take, rank, jnp.int32(max_num))].set(s2, mode='drop')
    j = jnp.arange(max_num, dtype=jnp.int32)
    appended = jnp.where(j < sel_len1, set_list[j],
                         uniq[jnp.clip(j - sel_len1, 0, max_num - 1)])
    seeds = appended[None, :]

    return confidence, corr_feats, M, H2d[None], seeds
